# Initial kernel scaffold; baseline (speedup 1.0000x reference)
import jax
import jax.numpy as jnp
from jax.experimental import pallas as pl


def kernel(images, img_coords, W):
    raise NotImplementedError("write your pallas kernel here")


# speedup vs baseline: 19.4849x; 19.4849x over previous
import functools

import jax
import jax.numpy as jnp
from jax.experimental import pallas as pl
from jax.experimental.pallas import tpu as pltpu

_N = 4096
_D_IN = 2048
_D_FEAT = 512
_K = 8
_RB = 256
_PB = 512


def _proj_kernel(x_ref, w_ref, o_ref):
    o_ref[...] = jax.lax.dot_general(
        x_ref[...], w_ref[...], (((1,), (0,)), ((), ())),
        preferred_element_type=jnp.float32)


def _project(images, w):
    return pl.pallas_call(
        _proj_kernel,
        grid=(_N // _PB,),
        in_specs=[
            pl.BlockSpec((_PB, _D_IN), lambda i: (i, 0)),
            pl.BlockSpec((_D_IN, _D_FEAT), lambda i: (0, 0)),
        ],
        out_specs=pl.BlockSpec((_PB, _D_FEAT), lambda i: (i, 0)),
        out_shape=jax.ShapeDtypeStruct((_N, _D_FEAT), jnp.float32),
    )(images, w)


def _knn_kernel(pts_ref, nbr_ref, sq_ref):
    i = pl.program_id(0)
    pts = pts_ref[...]

    @pl.when(i == 0)
    def _():
        s = jnp.sum(pts * pts, axis=1, keepdims=True)
        sq_ref[...] = jax.lax.transpose(s, (1, 0))

    sq = sq_ref[...]
    q = pts_ref[pl.ds(i * _RB, _RB), :]
    qsq = jnp.sum(q * q, axis=1, keepdims=True)
    mm = jax.lax.dot_general(
        q, pts, (((1,), (1,)), ((), ())),
        preferred_element_type=jnp.float32)
    d2 = qsq + sq - 2.0 * mm

    cols = jax.lax.broadcasted_iota(jnp.int32, (_RB, _N), 1)
    kcols = jax.lax.broadcasted_iota(jnp.int32, (_RB, _K), 1)
    nbr = jnp.zeros((_RB, _K), jnp.int32)
    for t in range(_K + 1):
        mv = jnp.min(d2, axis=1, keepdims=True)
        idx = jnp.min(jnp.where(d2 == mv, cols, _N), axis=1,
                      keepdims=True)
        if t > 0:
            nbr = jnp.where(kcols == (t - 1), idx, nbr)
        d2 = jnp.where(cols == idx, jnp.inf, d2)
    nbr_ref[...] = nbr


def _knn_neighbors(pts):
    d = pts.shape[1]
    return pl.pallas_call(
        _knn_kernel,
        grid=(_N // _RB,),
        in_specs=[pl.BlockSpec((_N, d), lambda i: (0, 0))],
        out_specs=pl.BlockSpec((_RB, _K), lambda i: (i, 0)),
        out_shape=jax.ShapeDtypeStruct((_N, _K), jnp.int32),
        scratch_shapes=[pltpu.VMEM((1, _N), jnp.float32)],
    )(pts)


def kernel(images, img_coords, W):
    features = _project(images, W)
    nbr_spatial = _knn_neighbors(img_coords)
    nbr_latent = _knn_neighbors(features)
    src = jnp.repeat(jnp.arange(_N, dtype=jnp.int32), _K)
    edge_spatial = jnp.stack([src, nbr_spatial.reshape(-1)], axis=0)
    edge_latent = jnp.stack([src, nbr_latent.reshape(-1)], axis=0)
    return features, edge_spatial, edge_latent, img_coords
